# speedup vs baseline: 1.0236x; 1.0236x over previous
import functools
import math

import jax
import jax.numpy as jnp
from jax import lax
from jax.experimental import pallas as pl
from jax.experimental.pallas import tpu as pltpu
from jax.experimental.pallas import tpu_sc as plsc

VOCAB = 1_000_000
D = 128
B, L = 4096, 200
N = B * L
NC, NS = 2, 16
NW = NC * NS
PER_W = N // NW
G = 128
GROUPS = PER_W // G
SCALE = float(math.sqrt(128.0))

_mesh = plsc.VectorSubcoreMesh(core_axis_name="c", subcore_axis_name="s")


@functools.partial(
    pl.kernel,
    mesh=_mesh,
    out_type=jax.ShapeDtypeStruct((N, D), jnp.float32),
    scratch_types=[
        pltpu.VMEM((GROUPS, G), jnp.int32),
        pltpu.VMEM((G, D), jnp.float32),
        pltpu.SemaphoreType.DMA,
    ],
)
def _embed_sc(idx_hbm, table_hbm, out_hbm, idx_v, rows_v, sem):
    wid = lax.axis_index("s") * NC + lax.axis_index("c")
    row_base = wid * GROUPS
    pltpu.sync_copy(idx_hbm.at[pl.ds(row_base, GROUPS)], idx_v)

    def group_body(g, carry):
        pltpu.async_copy(table_hbm.at[idx_v.at[g]], rows_v, sem).wait()

        def scale_row(r, c2):
            for c in range(D // 16):
                sl = pl.ds(c * 16, 16)
                rows_v[r, sl] = rows_v[r, sl] * SCALE
            return c2

        lax.fori_loop(0, G, scale_row, 0)
        pltpu.sync_copy(rows_v, out_hbm.at[pl.ds((row_base + g) * G, G)])
        return carry

    lax.fori_loop(0, GROUPS, group_body, 0)


def kernel(x, input_embedding):
    idx = x.astype(jnp.int32).reshape(N // G, G)
    out = _embed_sc(idx, input_embedding)
    return out.reshape(B, L, D)


# speedup vs baseline: 1.8605x; 1.8177x over previous
import functools
import math

import jax
import jax.numpy as jnp
from jax import lax
from jax.experimental import pallas as pl
from jax.experimental.pallas import tpu as pltpu
from jax.experimental.pallas import tpu_sc as plsc

VOCAB = 1_000_000
D = 128
B, L = 4096, 200
N = B * L
NC, NS = 2, 16
NW = NC * NS
PER_W = N // NW
G = 128
GROUPS = PER_W // G
NBUF = 4
P = 2
SCALE = float(math.sqrt(128.0))

_mesh = plsc.VectorSubcoreMesh(core_axis_name="c", subcore_axis_name="s")


@functools.partial(
    pl.kernel,
    mesh=_mesh,
    out_type=jax.ShapeDtypeStruct((N, D), jnp.float32),
    scratch_types=[
        pltpu.VMEM((GROUPS, G), jnp.int32),
        pltpu.VMEM((NBUF, G, D), jnp.float32),
        pltpu.SemaphoreType.DMA((NBUF,)),
        pltpu.SemaphoreType.DMA((NBUF,)),
    ],
)
def _embed_sc(idx_hbm, table_hbm, out_hbm, idx_v, rows_v, gsem, ssem):
    wid = lax.axis_index("s") * NC + lax.axis_index("c")
    row_base = wid * GROUPS
    pltpu.sync_copy(idx_hbm.at[pl.ds(row_base, GROUPS)], idx_v)

    def start_gather(g, b):
        pltpu.async_copy(table_hbm.at[idx_v.at[g]], rows_v.at[b], gsem.at[b])

    def wait_gather(g, b):
        pltpu.make_async_copy(table_hbm.at[idx_v.at[g]], rows_v.at[b],
                              gsem.at[b]).wait()

    def start_scatter(g, b):
        pltpu.async_copy(rows_v.at[b],
                         out_hbm.at[pl.ds((row_base + g) * G, G)],
                         ssem.at[b])

    def wait_scatter(g, b):
        pltpu.make_async_copy(rows_v.at[b],
                              out_hbm.at[pl.ds((row_base + g) * G, G)],
                              ssem.at[b]).wait()

    def scale(b):
        def scale_row(r, c2):
            for c in range(D // 16):
                sl = pl.ds(c * 16, 16)
                rows_v[b, r, sl] = rows_v[b, r, sl] * SCALE
            return c2

        lax.fori_loop(0, G, scale_row, 0)

    def step(g, b, wait_sct, pref):
        wait_gather(g, b)
        scale(b)
        start_scatter(g, b)
        if wait_sct:
            wait_scatter(g + P - NBUF, (b + P) % NBUF)
        if pref:
            start_gather(g + P, (b + P) % NBUF)

    for g in range(P):
        start_gather(g, g % NBUF)
    for g in range(NBUF):
        step(g, g % NBUF, wait_sct=(g + P >= NBUF), pref=True)

    def outer(k, carry):
        for b in range(NBUF):
            g = k * NBUF + b
            step(g, b, wait_sct=True, pref=True)
        return carry

    lax.fori_loop(1, GROUPS // NBUF - 1, outer, 0)

    for g in range(GROUPS - NBUF, GROUPS):
        step(g, g % NBUF, wait_sct=True, pref=(g + P < GROUPS))
    for g in range(GROUPS - P, GROUPS):
        wait_scatter(g, g % NBUF)


def kernel(x, input_embedding):
    idx = x.astype(jnp.int32).reshape(N // G, G)
    out = _embed_sc(idx, input_embedding)
    return out.reshape(B, L, D)
